# baseline (speedup 1.0000x reference)
import functools

import jax
import jax.numpy as jnp
from jax.experimental import pallas as pl
from jax.experimental.pallas import tpu as pltpu

_TEMP_INV = 5.0
_TEMP_INV_LOG2E = 7.213475204444817
_ALPHA = 0.5
_GATE_THRESHOLD = 0.1


def _vq_tile(dim, x_ref, p_ref, s0_ref, win_ref, conf_ref, maxraw_ref,
             probs_ref, raw_ref):
    p = p_ref[...]
    n_probes = p.shape[0]
    pad = p.shape[1] - dim

    x = x_ref[...]
    cb = x.shape[0]

    inv_norm = 1.0 / jnp.maximum(
        jnp.sqrt(jnp.sum(x * x, axis=1, keepdims=True)), 1e-12)
    xn = x * inv_norm
    xn_pad = jnp.concatenate(
        [xn, jnp.zeros((cb, pad), jnp.float32)], axis=1)

    raw = jax.lax.dot_general(
        xn_pad, p, (((1,), (1,)), ((), ())),
        preferred_element_type=jnp.float32)
    raw_ref[...] = raw

    mraw = jnp.max(raw, axis=1, keepdims=True)
    maxraw_ref[0] = mraw

    d = raw - mraw
    e = jnp.exp2(d * _TEMP_INV_LOG2E)

    weighted_aug = jax.lax.dot_general(
        e, p, (((1,), (0,)), ((), ())),
        preferred_element_type=jnp.float32)
    weighted = weighted_aug[:, :dim]
    s = weighted_aug[:, dim:dim + 1]
    inv_s = 1.0 / s

    probs_ref[...] = e * inv_s
    conf_ref[0] = inv_s

    avg = jnp.sum(xn * weighted, axis=1, keepdims=True) * inv_s
    gate = jax.nn.sigmoid((avg - _GATE_THRESHOLD) * 10.0)
    s0_ref[...] = (_ALPHA * x + (1.0 - _ALPHA) * weighted * inv_s) * gate

    lanes = jax.lax.broadcasted_iota(
        jnp.int32, raw.shape, 1).astype(jnp.float32)
    win_ref[0] = jnp.min(
        jnp.where(d == 0.0, lanes, float(n_probes)),
        axis=1, keepdims=True).astype(jnp.int32)


def _vq_tile_pipe(x_ref, xp_ref, p_ref, s0_ref, win_ref, conf_ref,
                  maxraw_ref, probs_ref, raw_ref, buf0_ref, buf1_ref):
    i = pl.program_id(0)

    def body(bufw_ref, bufr_ref):
        p = p_ref[...]
        n_probes = p.shape[0]

        x = x_ref[...]
        inv_norm = 1.0 / jnp.maximum(
            jnp.sqrt(jnp.sum(x * x, axis=1, keepdims=True)), 1e-12)
        xn = x * inv_norm
        raw_i = jax.lax.dot_general(
            xn, p, (((1,), (1,)), ((), ())),
            preferred_element_type=jnp.float32)
        raw_ref[...] = raw_i
        bufw_ref[...] = raw_i

        raw = bufr_ref[...]

        mraw = jnp.max(raw, axis=1, keepdims=True)
        maxraw_ref[0] = mraw

        e = jnp.exp2((raw - mraw) * _TEMP_INV_LOG2E)
        s = jnp.sum(e, axis=1, keepdims=True)
        inv_s = 1.0 / s
        probs_ref[...] = e * inv_s
        conf_ref[0] = inv_s

        weighted = jax.lax.dot_general(
            e, p, (((1,), (0,)), ((), ())),
            preferred_element_type=jnp.float32)

        xp = xp_ref[...]
        inv_norm_p = 1.0 / jnp.maximum(
            jnp.sqrt(jnp.sum(xp * xp, axis=1, keepdims=True)), 1e-12)
        avg = jnp.sum((xp * inv_norm_p) * weighted,
                      axis=1, keepdims=True) * inv_s
        gate = jax.nn.sigmoid((avg - _GATE_THRESHOLD) * 10.0)
        s0_ref[...] = (_ALPHA * xp + (1.0 - _ALPHA) * weighted * inv_s) * gate

        lanes = jax.lax.broadcasted_iota(
            jnp.int32, raw.shape, 1).astype(jnp.float32)
        win_ref[0] = jnp.min(
            jnp.where(raw == mraw, lanes, float(n_probes)),
            axis=1, keepdims=True).astype(jnp.int32)

    @pl.when(jax.lax.rem(i, 2) == 0)
    def _():
        body(buf0_ref, buf1_ref)

    @pl.when(jax.lax.rem(i, 2) == 1)
    def _():
        body(buf1_ref, buf0_ref)


@functools.partial(jax.jit, static_argnames=("block_b",))
def _vq_call_pipe(x_input, probes, block_b=128):
    batch, dim = x_input.shape
    n_probes = probes.shape[0]
    nb = batch // block_b

    out_shapes = (
        jax.ShapeDtypeStruct((batch, dim), jnp.float32),
        jax.ShapeDtypeStruct((nb, block_b, 1), jnp.int32),
        jax.ShapeDtypeStruct((nb, block_b, 1), jnp.float32),
        jax.ShapeDtypeStruct((nb, block_b, 1), jnp.float32),
        jax.ShapeDtypeStruct((batch, n_probes), jnp.float32),
        jax.ShapeDtypeStruct((batch, n_probes), jnp.float32),
    )
    prev = lambda i: jnp.maximum(i - 1, 0)
    out_specs = (
        pl.BlockSpec((block_b, dim), lambda i: (prev(i), 0)),
        pl.BlockSpec((1, block_b, 1), lambda i: (prev(i), 0, 0)),
        pl.BlockSpec((1, block_b, 1), lambda i: (prev(i), 0, 0)),
        pl.BlockSpec((1, block_b, 1), lambda i: (prev(i), 0, 0)),
        pl.BlockSpec((block_b, n_probes), lambda i: (prev(i), 0)),
        pl.BlockSpec((block_b, n_probes),
                     lambda i: (jnp.minimum(i, nb - 1), 0)),
    )
    in_specs = (
        pl.BlockSpec((block_b, dim), lambda i: (jnp.minimum(i, nb - 1), 0)),
        pl.BlockSpec((block_b, dim), lambda i: (prev(i), 0)),
        pl.BlockSpec((n_probes, dim), lambda i: (0, 0)),
    )
    return pl.pallas_call(
        _vq_tile_pipe,
        grid=(nb + 1,),
        in_specs=in_specs,
        out_specs=out_specs,
        out_shape=out_shapes,
        scratch_shapes=[pltpu.VMEM((block_b, n_probes), jnp.float32),
                        pltpu.VMEM((block_b, n_probes), jnp.float32)],
        compiler_params=pltpu.CompilerParams(
            dimension_semantics=("arbitrary",),
            vmem_limit_bytes=100 * 1024 * 1024),
    )(x_input, x_input, probes)


@functools.partial(jax.jit, static_argnames=("block_b",))
def _vq_call(x_input, probes, block_b=256):
    batch, dim = x_input.shape
    n_probes = probes.shape[0]
    nb = batch // block_b
    p_aug = jnp.concatenate(
        [probes,
         jnp.ones((n_probes, 1), jnp.float32),
         jnp.zeros((n_probes, 127), jnp.float32)], axis=1)

    out_shapes = (
        jax.ShapeDtypeStruct((batch, dim), jnp.float32),
        jax.ShapeDtypeStruct((nb, block_b, 1), jnp.int32),
        jax.ShapeDtypeStruct((nb, block_b, 1), jnp.float32),
        jax.ShapeDtypeStruct((nb, block_b, 1), jnp.float32),
        jax.ShapeDtypeStruct((batch, n_probes), jnp.float32),
        jax.ShapeDtypeStruct((batch, n_probes), jnp.float32),
    )
    out_specs = (
        pl.BlockSpec((block_b, dim), lambda i: (i, 0)),
        pl.BlockSpec((1, block_b, 1), lambda i: (i, 0, 0)),
        pl.BlockSpec((1, block_b, 1), lambda i: (i, 0, 0)),
        pl.BlockSpec((1, block_b, 1), lambda i: (i, 0, 0)),
        pl.BlockSpec((block_b, n_probes), lambda i: (i, 0)),
        pl.BlockSpec((block_b, n_probes), lambda i: (i, 0)),
    )
    in_specs = (
        pl.BlockSpec((block_b, dim), lambda i: (i, 0)),
        pl.BlockSpec((n_probes, dim + 128), lambda i: (0, 0)),
    )
    return pl.pallas_call(
        functools.partial(_vq_tile, dim),
        grid=(nb,),
        in_specs=in_specs,
        out_specs=out_specs,
        out_shape=out_shapes,
        compiler_params=pltpu.CompilerParams(
            dimension_semantics=("parallel",),
            vmem_limit_bytes=63 * 1024 * 1024),
    )(x_input, p_aug)


def kernel(x_input, probes):
    batch = x_input.shape[0]
    s0, win, conf, maxraw, probs, raw = _vq_call(
        x_input, probes, block_b=min(256, batch))
    s0 = s0.reshape(batch, x_input.shape[1])
    win = win.reshape(batch)
    conf = conf.reshape(batch)
    maxraw = maxraw.reshape(batch)
    gate_open = maxraw > _GATE_THRESHOLD
    return (s0, win, conf, maxraw, gate_open, probs, raw)


# speedup vs baseline: 1.0503x; 1.0503x over previous
import functools

import jax
import jax.numpy as jnp
from jax.experimental import pallas as pl
from jax.experimental.pallas import tpu as pltpu

_TEMP_INV = 5.0
_TEMP_INV_LOG2E = 7.213475204444817
_ALPHA = 0.5
_GATE_THRESHOLD = 0.1


def _vq_tile(n_chunks, x_ref, p_ref, s0_ref, win_ref, conf_ref, maxraw_ref,
             probs_ref, raw_ref):
    p = p_ref[...]
    n_probes = p.shape[0]
    cb = x_ref.shape[0] // n_chunks

    for c in range(n_chunks):
        r = pl.ds(c * cb, cb)
        x = x_ref[r, :]

        inv_norm = 1.0 / jnp.maximum(
            jnp.sqrt(jnp.sum(x * x, axis=1, keepdims=True)), 1e-12)
        xn = x * inv_norm

        raw = jax.lax.dot_general(
            xn, p, (((1,), (1,)), ((), ())),
            preferred_element_type=jnp.float32)
        raw_ref[r, :] = raw

        mraw = jnp.max(raw, axis=1, keepdims=True)
        maxraw_ref[0, r, :] = mraw

        e = jnp.exp2((raw - mraw) * _TEMP_INV_LOG2E)
        s = jnp.sum(e, axis=1, keepdims=True)
        inv_s = 1.0 / s
        probs_ref[r, :] = e * inv_s
        conf_ref[0, r, :] = inv_s

        weighted = jax.lax.dot_general(
            e, p, (((1,), (0,)), ((), ())),
            preferred_element_type=jnp.float32)

        avg = jnp.sum(xn * weighted, axis=1, keepdims=True) * inv_s
        gate = jax.nn.sigmoid((avg - _GATE_THRESHOLD) * 10.0)
        s0_ref[r, :] = (_ALPHA * x + (1.0 - _ALPHA) * weighted * inv_s) * gate

        lanes = jax.lax.broadcasted_iota(
            jnp.int32, raw.shape, 1).astype(jnp.float32)
        win_ref[0, r, :] = jnp.min(
            jnp.where(raw == mraw, lanes, float(n_probes)),
            axis=1, keepdims=True).astype(jnp.int32)


def _vq_tile_pipe(nb, x_ref, xp_ref, p_ref, s0_ref, win_ref, conf_ref,
                  maxraw_ref, probs_ref, raw_ref, rawbuf_ref):
    i = pl.program_id(0)
    jw = jax.lax.rem(i, 2)
    jr = jax.lax.rem(i + 1, 2)
    p = p_ref[...]
    n_probes = p.shape[0]

    x = x_ref[...]
    inv_norm = 1.0 / jnp.maximum(
        jnp.sqrt(jnp.sum(x * x, axis=1, keepdims=True)), 1e-12)
    xn = x * inv_norm
    rawbuf_ref[jw] = jax.lax.dot_general(
        xn, p, (((1,), (1,)), ((), ())), preferred_element_type=jnp.float32)

    raw = rawbuf_ref[jr]
    raw_ref[...] = raw

    mraw = jnp.max(raw, axis=1, keepdims=True)
    maxraw_ref[0] = mraw

    e = jnp.exp2((raw - mraw) * _TEMP_INV_LOG2E)
    s = jnp.sum(e, axis=1, keepdims=True)
    inv_s = 1.0 / s
    probs_ref[...] = e * inv_s
    conf_ref[0] = inv_s

    weighted = jax.lax.dot_general(
        e, p, (((1,), (0,)), ((), ())), preferred_element_type=jnp.float32)

    xp = xp_ref[...]
    avg = jnp.sum(raw * e, axis=1, keepdims=True) * inv_s
    gate = jax.nn.sigmoid((avg - _GATE_THRESHOLD) * 10.0)
    s0_ref[...] = (_ALPHA * xp + (1.0 - _ALPHA) * weighted * inv_s) * gate

    lanes = jax.lax.broadcasted_iota(
        jnp.int32, raw.shape, 1).astype(jnp.float32)
    win_ref[0] = jnp.min(
        jnp.where(raw == mraw, lanes, float(n_probes)),
        axis=1, keepdims=True).astype(jnp.int32)


@functools.partial(jax.jit, static_argnames=("block_b",))
def _vq_call_pipe(x_input, probes, block_b=128):
    batch, dim = x_input.shape
    n_probes = probes.shape[0]
    nb = batch // block_b

    out_shapes = (
        jax.ShapeDtypeStruct((batch, dim), jnp.float32),
        jax.ShapeDtypeStruct((nb, block_b, 1), jnp.int32),
        jax.ShapeDtypeStruct((nb, block_b, 1), jnp.float32),
        jax.ShapeDtypeStruct((nb, block_b, 1), jnp.float32),
        jax.ShapeDtypeStruct((batch, n_probes), jnp.float32),
        jax.ShapeDtypeStruct((batch, n_probes), jnp.float32),
    )
    prev = lambda i: jnp.maximum(i - 1, 0)
    out_specs = (
        pl.BlockSpec((block_b, dim), lambda i: (prev(i), 0)),
        pl.BlockSpec((1, block_b, 1), lambda i: (prev(i), 0, 0)),
        pl.BlockSpec((1, block_b, 1), lambda i: (prev(i), 0, 0)),
        pl.BlockSpec((1, block_b, 1), lambda i: (prev(i), 0, 0)),
        pl.BlockSpec((block_b, n_probes), lambda i: (prev(i), 0)),
        pl.BlockSpec((block_b, n_probes), lambda i: (prev(i), 0)),
    )
    in_specs = (
        pl.BlockSpec((block_b, dim), lambda i: (jnp.minimum(i, nb - 1), 0)),
        pl.BlockSpec((block_b, dim), lambda i: (prev(i), 0)),
        pl.BlockSpec((n_probes, dim), lambda i: (0, 0)),
    )
    return pl.pallas_call(
        functools.partial(_vq_tile_pipe, nb),
        grid=(nb + 1,),
        in_specs=in_specs,
        out_specs=out_specs,
        out_shape=out_shapes,
        scratch_shapes=[pltpu.VMEM((2, block_b, n_probes), jnp.float32)],
        compiler_params=pltpu.CompilerParams(
            dimension_semantics=("arbitrary",)),
    )(x_input, x_input, probes)


@functools.partial(jax.jit, static_argnames=("block_b", "n_chunks"))
def _vq_call(x_input, probes, block_b=256, n_chunks=1):
    batch, dim = x_input.shape
    n_probes = probes.shape[0]
    nb = batch // block_b

    out_shapes = (
        jax.ShapeDtypeStruct((batch, dim), jnp.float32),
        jax.ShapeDtypeStruct((nb, block_b, 1), jnp.int32),
        jax.ShapeDtypeStruct((nb, block_b, 1), jnp.float32),
        jax.ShapeDtypeStruct((nb, block_b, 1), jnp.float32),
        jax.ShapeDtypeStruct((batch, n_probes), jnp.float32),
        jax.ShapeDtypeStruct((batch, n_probes), jnp.float32),
    )
    out_specs = (
        pl.BlockSpec((block_b, dim), lambda i: (i, 0)),
        pl.BlockSpec((1, block_b, 1), lambda i: (i, 0, 0)),
        pl.BlockSpec((1, block_b, 1), lambda i: (i, 0, 0)),
        pl.BlockSpec((1, block_b, 1), lambda i: (i, 0, 0)),
        pl.BlockSpec((block_b, n_probes), lambda i: (i, 0)),
        pl.BlockSpec((block_b, n_probes), lambda i: (i, 0)),
    )
    in_specs = (
        pl.BlockSpec((block_b, dim), lambda i: (i, 0)),
        pl.BlockSpec((n_probes, dim), lambda i: (0, 0)),
    )
    return pl.pallas_call(
        functools.partial(_vq_tile, n_chunks),
        grid=(nb,),
        in_specs=in_specs,
        out_specs=out_specs,
        out_shape=out_shapes,
        compiler_params=pltpu.CompilerParams(
            dimension_semantics=("parallel",)),
    )(x_input, probes)


def kernel(x_input, probes):
    batch = x_input.shape[0]
    s0, win, conf, maxraw, probs, raw = _vq_call(
        x_input, probes, block_b=min(256, batch),
        n_chunks=2 if batch % 256 == 0 else 1)
    s0 = s0.reshape(batch, x_input.shape[1])
    win = win.reshape(batch)
    conf = conf.reshape(batch)
    maxraw = maxraw.reshape(batch)
    gate_open = maxraw > _GATE_THRESHOLD
    return (s0, win, conf, maxraw, gate_open, probs, raw)


# speedup vs baseline: 1.0844x; 1.0324x over previous
import functools

import jax
import jax.numpy as jnp
from jax.experimental import pallas as pl
from jax.experimental.pallas import tpu as pltpu

_TEMP_INV = 5.0
_TEMP_INV_LOG2E = 7.213475204444817
_ALPHA = 0.5
_GATE_THRESHOLD = 0.1


def _vq_tile(n_chunks, x_ref, p_ref, s0_ref, win_ref, conf_ref, maxraw_ref,
             probs_ref, raw_ref):
    p = p_ref[...]
    n_probes = p.shape[0]
    cb = x_ref.shape[0] // n_chunks

    for c in range(n_chunks):
        r = pl.ds(c * cb, cb)
        x = x_ref[r, :]

        inv_norm = 1.0 / jnp.maximum(
            jnp.sqrt(jnp.sum(x * x, axis=1, keepdims=True)), 1e-12)
        xn = x * inv_norm

        raw = jax.lax.dot_general(
            xn, p, (((1,), (1,)), ((), ())),
            preferred_element_type=jnp.float32)
        raw_ref[r, :] = raw

        mraw = jnp.max(raw, axis=1, keepdims=True)
        maxraw_ref[0, r, :] = mraw

        e = jnp.exp2((raw - mraw) * _TEMP_INV_LOG2E)
        s = jnp.sum(e, axis=1, keepdims=True)
        inv_s = 1.0 / s
        probs_ref[r, :] = e * inv_s
        conf_ref[0, r, :] = inv_s

        weighted = jax.lax.dot_general(
            e, p, (((1,), (0,)), ((), ())),
            preferred_element_type=jnp.float32)

        avg = jnp.sum(xn * weighted, axis=1, keepdims=True) * inv_s
        gate = jax.nn.sigmoid((avg - _GATE_THRESHOLD) * 10.0)
        s0_ref[r, :] = (_ALPHA * x + (1.0 - _ALPHA) * weighted * inv_s) * gate

        lanes = jax.lax.broadcasted_iota(
            jnp.int32, raw.shape, 1).astype(jnp.float32)
        win_ref[0, r, :] = jnp.min(
            jnp.where(raw == mraw, lanes, float(n_probes)),
            axis=1, keepdims=True).astype(jnp.int32)


def _vq_tile_pipe(nb, x_ref, xp_ref, p_ref, s0_ref, win_ref, conf_ref,
                  maxraw_ref, probs_ref, raw_ref, rawbuf_ref):
    i = pl.program_id(0)
    jw = jax.lax.rem(i, 2)
    jr = jax.lax.rem(i + 1, 2)
    p = p_ref[...]
    n_probes = p.shape[0]

    x = x_ref[...]
    inv_norm = 1.0 / jnp.maximum(
        jnp.sqrt(jnp.sum(x * x, axis=1, keepdims=True)), 1e-12)
    xn = x * inv_norm
    rawbuf_ref[jw] = jax.lax.dot_general(
        xn, p, (((1,), (1,)), ((), ())), preferred_element_type=jnp.float32)

    raw = rawbuf_ref[jr]
    raw_ref[...] = raw

    mraw = jnp.max(raw, axis=1, keepdims=True)
    maxraw_ref[0] = mraw

    e = jnp.exp2((raw - mraw) * _TEMP_INV_LOG2E)
    s = jnp.sum(e, axis=1, keepdims=True)
    inv_s = 1.0 / s
    probs_ref[...] = e * inv_s
    conf_ref[0] = inv_s

    weighted = jax.lax.dot_general(
        e, p, (((1,), (0,)), ((), ())), preferred_element_type=jnp.float32)

    xp = xp_ref[...]
    avg = jnp.sum(raw * e, axis=1, keepdims=True) * inv_s
    gate = jax.nn.sigmoid((avg - _GATE_THRESHOLD) * 10.0)
    s0_ref[...] = (_ALPHA * xp + (1.0 - _ALPHA) * weighted * inv_s) * gate

    lanes = jax.lax.broadcasted_iota(
        jnp.int32, raw.shape, 1).astype(jnp.float32)
    win_ref[0] = jnp.min(
        jnp.where(raw == mraw, lanes, float(n_probes)),
        axis=1, keepdims=True).astype(jnp.int32)


@functools.partial(jax.jit, static_argnames=("block_b",))
def _vq_call_pipe(x_input, probes, block_b=128):
    batch, dim = x_input.shape
    n_probes = probes.shape[0]
    nb = batch // block_b

    out_shapes = (
        jax.ShapeDtypeStruct((batch, dim), jnp.float32),
        jax.ShapeDtypeStruct((nb, block_b, 1), jnp.int32),
        jax.ShapeDtypeStruct((nb, block_b, 1), jnp.float32),
        jax.ShapeDtypeStruct((nb, block_b, 1), jnp.float32),
        jax.ShapeDtypeStruct((batch, n_probes), jnp.float32),
        jax.ShapeDtypeStruct((batch, n_probes), jnp.float32),
    )
    prev = lambda i: jnp.maximum(i - 1, 0)
    out_specs = (
        pl.BlockSpec((block_b, dim), lambda i: (prev(i), 0)),
        pl.BlockSpec((1, block_b, 1), lambda i: (prev(i), 0, 0)),
        pl.BlockSpec((1, block_b, 1), lambda i: (prev(i), 0, 0)),
        pl.BlockSpec((1, block_b, 1), lambda i: (prev(i), 0, 0)),
        pl.BlockSpec((block_b, n_probes), lambda i: (prev(i), 0)),
        pl.BlockSpec((block_b, n_probes), lambda i: (prev(i), 0)),
    )
    in_specs = (
        pl.BlockSpec((block_b, dim), lambda i: (jnp.minimum(i, nb - 1), 0)),
        pl.BlockSpec((block_b, dim), lambda i: (prev(i), 0)),
        pl.BlockSpec((n_probes, dim), lambda i: (0, 0)),
    )
    return pl.pallas_call(
        functools.partial(_vq_tile_pipe, nb),
        grid=(nb + 1,),
        in_specs=in_specs,
        out_specs=out_specs,
        out_shape=out_shapes,
        scratch_shapes=[pltpu.VMEM((2, block_b, n_probes), jnp.float32)],
        compiler_params=pltpu.CompilerParams(
            dimension_semantics=("arbitrary",)),
    )(x_input, x_input, probes)


@functools.partial(jax.jit, static_argnames=("block_b", "n_chunks"))
def _vq_call(x_input, probes, block_b=256, n_chunks=1):
    batch, dim = x_input.shape
    n_probes = probes.shape[0]
    nb = batch // block_b

    out_shapes = (
        jax.ShapeDtypeStruct((batch, dim), jnp.float32),
        jax.ShapeDtypeStruct((nb, block_b, 1), jnp.int32),
        jax.ShapeDtypeStruct((nb, block_b, 1), jnp.float32),
        jax.ShapeDtypeStruct((nb, block_b, 1), jnp.float32),
        jax.ShapeDtypeStruct((batch, n_probes), jnp.float32),
        jax.ShapeDtypeStruct((batch, n_probes), jnp.float32),
    )
    out_specs = (
        pl.BlockSpec((block_b, dim), lambda i: (i, 0)),
        pl.BlockSpec((1, block_b, 1), lambda i: (i, 0, 0)),
        pl.BlockSpec((1, block_b, 1), lambda i: (i, 0, 0)),
        pl.BlockSpec((1, block_b, 1), lambda i: (i, 0, 0)),
        pl.BlockSpec((block_b, n_probes), lambda i: (i, 0)),
        pl.BlockSpec((block_b, n_probes), lambda i: (i, 0)),
    )
    in_specs = (
        pl.BlockSpec((block_b, dim), lambda i: (i, 0)),
        pl.BlockSpec((n_probes, dim), lambda i: (0, 0)),
    )
    return pl.pallas_call(
        functools.partial(_vq_tile, n_chunks),
        grid=(nb,),
        in_specs=in_specs,
        out_specs=out_specs,
        out_shape=out_shapes,
        compiler_params=pltpu.CompilerParams(
            dimension_semantics=("parallel",),
            vmem_limit_bytes=63 * 1024 * 1024),
    )(x_input, probes)


def kernel(x_input, probes):
    batch = x_input.shape[0]
    s0, win, conf, maxraw, probs, raw = _vq_call(
        x_input, probes, block_b=min(256, batch),
        n_chunks=1)
    s0 = s0.reshape(batch, x_input.shape[1])
    win = win.reshape(batch)
    conf = conf.reshape(batch)
    maxraw = maxraw.reshape(batch)
    gate_open = maxraw > _GATE_THRESHOLD
    return (s0, win, conf, maxraw, gate_open, probs, raw)
